# Initial kernel scaffold; baseline (speedup 1.0000x reference)
import jax
import jax.numpy as jnp
from jax.experimental import pallas as pl


def kernel(x, Wr, br, W1, b1, W2, b2):
    raise NotImplementedError("write your pallas kernel here")


# speedup vs baseline: 2.9258x; 2.9258x over previous
import functools
import math

import jax
import jax.numpy as jnp
from jax.experimental import pallas as pl
from jax.experimental.pallas import tpu as pltpu


def _moe_body(x_ref, wr_ref, br_ref, w1_ref, b1_ref, w2_ref, b2_ref,
              out_ref, acc_ref, *, n_experts, tile_s):
    e = pl.program_id(0)
    i = pl.program_id(1)

    xb = x_ref[...]

    logits = jnp.dot(xb, wr_ref[...].astype(jnp.bfloat16),
                     preferred_element_type=jnp.float32)
    logits = logits + br_ref[...]
    logits = logits - jnp.max(logits, axis=-1, keepdims=True)
    p = jnp.exp(logits)
    weights = p / jnp.sum(p, axis=-1, keepdims=True)
    lane = jax.lax.broadcasted_iota(jnp.int32, weights.shape, 1)
    w_e = jnp.sum(jnp.where(lane == e, weights, 0.0), axis=1,
                  keepdims=True)

    h = jnp.dot(xb, w1_ref[0], preferred_element_type=jnp.float32)
    h = h + b1_ref[0]
    h = 0.5 * h * (1.0 + jax.lax.erf(h * (1.0 / math.sqrt(2.0))))
    o = jnp.dot(h.astype(jnp.bfloat16), w2_ref[0],
                preferred_element_type=jnp.float32)
    o = o + b2_ref[0]
    contrib = w_e * o

    sl = pl.ds(i * tile_s, tile_s)

    @pl.when(e == 0)
    def _():
        acc_ref[sl, :] = contrib

    @pl.when(e != 0)
    def _():
        acc_ref[sl, :] = acc_ref[sl, :] + contrib

    out_ref[...] = acc_ref[sl, :]


@functools.partial(jax.jit, static_argnames=())
def _soft_moe(x, Wr, br, W1, b1, W2, b2):
    B, S, D = x.shape
    E = Wr.shape[1]
    FF = W1.shape[2]
    TS = 256

    x2 = x.reshape(S, D).astype(jnp.bfloat16)
    w1b = W1.astype(jnp.bfloat16)
    w2b = W2.astype(jnp.bfloat16)
    br2 = br.reshape(1, E)
    b1r = b1.reshape(E, 1, FF)
    b2r = b2.reshape(E, 1, D)

    grid = (E, S // TS)
    out = pl.pallas_call(
        functools.partial(_moe_body, n_experts=E, tile_s=TS),
        grid=grid,
        in_specs=[
            pl.BlockSpec((TS, D), lambda e, i: (i, 0)),
            pl.BlockSpec((D, E), lambda e, i: (0, 0)),
            pl.BlockSpec((1, E), lambda e, i: (0, 0)),
            pl.BlockSpec((1, D, FF), lambda e, i: (e, 0, 0)),
            pl.BlockSpec((1, 1, FF), lambda e, i: (e, 0, 0)),
            pl.BlockSpec((1, FF, D), lambda e, i: (e, 0, 0)),
            pl.BlockSpec((1, 1, D), lambda e, i: (e, 0, 0)),
        ],
        out_specs=pl.BlockSpec((TS, D), lambda e, i: (i, 0)),
        out_shape=jax.ShapeDtypeStruct((S, D), jnp.float32),
        scratch_shapes=[pltpu.VMEM((S, D), jnp.float32)],
    )(x2, Wr, br2, w1b, b1r, w2b, b2r)
    return out.reshape(B, S, D)


def kernel(x, Wr, br, W1, b1, W2, b2):
    return _soft_moe(x, Wr, br, W1, b1, W2, b2)


# speedup vs baseline: 3.1942x; 1.0917x over previous
import functools
import math

import jax
import jax.numpy as jnp
from jax.experimental import pallas as pl
from jax.experimental.pallas import tpu as pltpu


def _moe_body(x_ref, wr_ref, br_ref, w1_ref, b1_ref, w2_ref, b2_ref,
              out_ref, acc_ref, *, n_experts, tile_s):
    e = pl.program_id(0)
    i = pl.program_id(1)

    xb = x_ref[...]

    logits = jnp.dot(xb, wr_ref[...].astype(jnp.bfloat16),
                     preferred_element_type=jnp.float32)
    logits = logits + br_ref[...]
    logits = logits - jnp.max(logits, axis=-1, keepdims=True)
    p = jnp.exp(logits)
    weights = p / jnp.sum(p, axis=-1, keepdims=True)
    lane = jax.lax.broadcasted_iota(jnp.int32, weights.shape, 1)
    w_e = jnp.sum(jnp.where(lane == e, weights, 0.0), axis=1,
                  keepdims=True)

    h = jnp.dot(xb, w1_ref[0], preferred_element_type=jnp.float32)
    h = h + b1_ref[0]
    h = 0.5 * h * (1.0 + jax.lax.erf(h * (1.0 / math.sqrt(2.0))))
    o = jnp.dot(h.astype(jnp.bfloat16), w2_ref[0],
                preferred_element_type=jnp.float32)
    o = o + b2_ref[0]
    contrib = w_e * o

    sl = pl.ds(i * tile_s, tile_s)

    @pl.when(e == 0)
    def _():
        acc_ref[sl, :] = contrib

    @pl.when(e != 0)
    def _():
        acc_ref[sl, :] = acc_ref[sl, :] + contrib

    out_ref[...] = acc_ref[sl, :]


@functools.partial(jax.jit, static_argnames=())
def _soft_moe(x, Wr, br, W1, b1, W2, b2):
    B, S, D = x.shape
    E = Wr.shape[1]
    FF = W1.shape[2]
    TS = 512

    x2 = x.reshape(S, D).astype(jnp.bfloat16)
    w1b = W1.astype(jnp.bfloat16)
    w2b = W2.astype(jnp.bfloat16)
    br2 = br.reshape(1, E)
    b1r = b1.reshape(E, 1, FF)
    b2r = b2.reshape(E, 1, D)

    grid = (E, S // TS)
    out = pl.pallas_call(
        functools.partial(_moe_body, n_experts=E, tile_s=TS),
        grid=grid,
        in_specs=[
            pl.BlockSpec((TS, D), lambda e, i: (i, 0)),
            pl.BlockSpec((D, E), lambda e, i: (0, 0)),
            pl.BlockSpec((1, E), lambda e, i: (0, 0)),
            pl.BlockSpec((1, D, FF), lambda e, i: (e, 0, 0)),
            pl.BlockSpec((1, 1, FF), lambda e, i: (e, 0, 0)),
            pl.BlockSpec((1, FF, D), lambda e, i: (e, 0, 0)),
            pl.BlockSpec((1, 1, D), lambda e, i: (e, 0, 0)),
        ],
        out_specs=pl.BlockSpec((TS, D), lambda e, i: (i, 0)),
        out_shape=jax.ShapeDtypeStruct((S, D), jnp.float32),
        scratch_shapes=[pltpu.VMEM((S, D), jnp.float32)],
    )(x2, Wr, br2, w1b, b1r, w2b, b2r)
    return out.reshape(B, S, D)


def kernel(x, Wr, br, W1, b1, W2, b2):
    return _soft_moe(x, Wr, br, W1, b1, W2, b2)


# speedup vs baseline: 3.2159x; 1.0068x over previous
import functools
import math

import jax
import jax.numpy as jnp
from jax.experimental import pallas as pl
from jax.experimental.pallas import tpu as pltpu


def _moe_body(x_ref, wr_ref, br_ref, w1_ref, b1_ref, w2_ref, b2_ref,
              out_ref, *, tile_s):
    e = pl.program_id(0)
    i = pl.program_id(1)
    sl = pl.ds(i * tile_s, tile_s)

    xb = x_ref[sl, :]

    logits = jnp.dot(xb, wr_ref[...].astype(jnp.bfloat16),
                     preferred_element_type=jnp.float32)
    logits = logits + br_ref[...]
    logits = logits - jnp.max(logits, axis=-1, keepdims=True)
    p = jnp.exp(logits)
    weights = p / jnp.sum(p, axis=-1, keepdims=True)
    lane = jax.lax.broadcasted_iota(jnp.int32, weights.shape, 1)
    w_e = jnp.sum(jnp.where(lane == e, weights, 0.0), axis=1,
                  keepdims=True)

    h = jnp.dot(xb, w1_ref[0], preferred_element_type=jnp.float32)
    h = h + b1_ref[0]
    h = 0.5 * h * (1.0 + jax.lax.erf(h * (1.0 / math.sqrt(2.0))))
    o = jnp.dot(h.astype(jnp.bfloat16), w2_ref[0],
                preferred_element_type=jnp.float32)
    o = o + b2_ref[0]
    contrib = w_e * o

    @pl.when(e == 0)
    def _():
        out_ref[sl, :] = contrib

    @pl.when(e != 0)
    def _():
        out_ref[sl, :] = out_ref[sl, :] + contrib


@jax.jit
def _soft_moe(x, Wr, br, W1, b1, W2, b2):
    B, S, D = x.shape
    E = Wr.shape[1]
    FF = W1.shape[2]
    TS = 512

    x2 = x.reshape(S, D).astype(jnp.bfloat16)
    w1b = W1.astype(jnp.bfloat16)
    w2b = W2.astype(jnp.bfloat16)
    br2 = br.reshape(1, E)
    b1r = b1.reshape(E, 1, FF)
    b2r = b2.reshape(E, 1, D)

    grid = (E, S // TS)
    out = pl.pallas_call(
        functools.partial(_moe_body, tile_s=TS),
        grid=grid,
        in_specs=[
            pl.BlockSpec((S, D), lambda e, i: (0, 0)),
            pl.BlockSpec((D, E), lambda e, i: (0, 0)),
            pl.BlockSpec((1, E), lambda e, i: (0, 0)),
            pl.BlockSpec((1, D, FF), lambda e, i: (e, 0, 0)),
            pl.BlockSpec((1, 1, FF), lambda e, i: (e, 0, 0)),
            pl.BlockSpec((1, FF, D), lambda e, i: (e, 0, 0)),
            pl.BlockSpec((1, 1, D), lambda e, i: (e, 0, 0)),
        ],
        out_specs=pl.BlockSpec((S, D), lambda e, i: (0, 0)),
        out_shape=jax.ShapeDtypeStruct((S, D), jnp.float32),
    )(x2, Wr, br2, w1b, b1r, w2b, b2r)
    return out.reshape(B, S, D)


def kernel(x, Wr, br, W1, b1, W2, b2):
    return _soft_moe(x, Wr, br, W1, b1, W2, b2)


# speedup vs baseline: 3.2432x; 1.0085x over previous
import functools
import math

import jax
import jax.numpy as jnp
from jax.experimental import pallas as pl
from jax.experimental.pallas import tpu as pltpu


def _moe_body(x_ref, wr_ref, br_ref, w1_ref, b1_ref, w2_ref, b2_ref,
              out_ref, *, tile_s, ff_chunk):
    e = pl.program_id(0)
    i = pl.program_id(1)
    sl = pl.ds(i * tile_s, tile_s)

    xb = x_ref[sl, :]

    logits = jnp.dot(xb, wr_ref[...].astype(jnp.bfloat16),
                     preferred_element_type=jnp.float32)
    logits = logits + br_ref[...]
    logits = logits - jnp.max(logits, axis=-1, keepdims=True)
    p = jnp.exp(logits)
    weights = p / jnp.sum(p, axis=-1, keepdims=True)
    lane = jax.lax.broadcasted_iota(jnp.int32, weights.shape, 1)
    w_e = jnp.sum(jnp.where(lane == e, weights, 0.0), axis=1,
                  keepdims=True)

    ff = w1_ref.shape[2]
    n_chunks = ff // ff_chunk
    o = b2_ref[0].astype(jnp.float32)
    for k in range(n_chunks):
        ks = slice(k * ff_chunk, (k + 1) * ff_chunk)
        h = jnp.dot(xb, w1_ref[0, :, ks],
                    preferred_element_type=jnp.float32)
        h = h + b1_ref[0, :, ks]
        h = 0.5 * h * (1.0 + jax.lax.erf(h * (1.0 / math.sqrt(2.0))))
        o = o + jnp.dot(h.astype(jnp.bfloat16), w2_ref[0, ks, :],
                        preferred_element_type=jnp.float32)
    contrib = w_e * o

    @pl.when(e == 0)
    def _():
        out_ref[sl, :] = contrib

    @pl.when(e != 0)
    def _():
        out_ref[sl, :] = out_ref[sl, :] + contrib


@jax.jit
def _soft_moe(x, Wr, br, W1, b1, W2, b2):
    B, S, D = x.shape
    E = Wr.shape[1]
    FF = W1.shape[2]
    TS = 512

    x2 = x.reshape(S, D).astype(jnp.bfloat16)
    w1b = W1.astype(jnp.bfloat16)
    w2b = W2.astype(jnp.bfloat16)
    br2 = br.reshape(1, E)
    b1r = b1.reshape(E, 1, FF)
    b2r = b2.reshape(E, 1, D)

    grid = (E, S // TS)
    out = pl.pallas_call(
        functools.partial(_moe_body, tile_s=TS, ff_chunk=FF // 4),
        grid=grid,
        in_specs=[
            pl.BlockSpec((S, D), lambda e, i: (0, 0)),
            pl.BlockSpec((D, E), lambda e, i: (0, 0)),
            pl.BlockSpec((1, E), lambda e, i: (0, 0)),
            pl.BlockSpec((1, D, FF), lambda e, i: (e, 0, 0)),
            pl.BlockSpec((1, 1, FF), lambda e, i: (e, 0, 0)),
            pl.BlockSpec((1, FF, D), lambda e, i: (e, 0, 0)),
            pl.BlockSpec((1, 1, D), lambda e, i: (e, 0, 0)),
        ],
        out_specs=pl.BlockSpec((S, D), lambda e, i: (0, 0)),
        out_shape=jax.ShapeDtypeStruct((S, D), jnp.float32),
    )(x2, Wr, br2, w1b, b1r, w2b, b2r)
    return out.reshape(B, S, D)


def kernel(x, Wr, br, W1, b1, W2, b2):
    return _soft_moe(x, Wr, br, W1, b1, W2, b2)


# speedup vs baseline: 4.2460x; 1.3092x over previous
import functools
import math

import jax
import jax.numpy as jnp
from jax.experimental import pallas as pl
from jax.experimental.pallas import tpu as pltpu


def _moe_body(x_ref, wr_ref, br_ref, w1_ref, b1_ref, w2_ref, b2_ref,
              out_ref, *, tile_s, ff_chunk):
    e = pl.program_id(0)
    i = pl.program_id(1)
    sl = pl.ds(i * tile_s, tile_s)

    xb = x_ref[sl, :]

    logits = jnp.dot(xb, wr_ref[...].astype(jnp.bfloat16),
                     preferred_element_type=jnp.float32)
    logits = logits + br_ref[...]
    logits = logits - jnp.max(logits, axis=-1, keepdims=True)
    p = jnp.exp(logits)
    weights = p / jnp.sum(p, axis=-1, keepdims=True)
    lane = jax.lax.broadcasted_iota(jnp.int32, weights.shape, 1)
    w_e = jnp.sum(jnp.where(lane == e, weights, 0.0), axis=1,
                  keepdims=True)

    ff = w1_ref.shape[2]
    n_chunks = ff // ff_chunk
    o = b2_ref[0].astype(jnp.float32)
    for k in range(n_chunks):
        ks = slice(k * ff_chunk, (k + 1) * ff_chunk)
        h = jnp.dot(xb, w1_ref[0, :, ks].astype(jnp.bfloat16),
                    preferred_element_type=jnp.float32)
        h = h + b1_ref[0, :, ks]
        h = 0.5 * h * (1.0 + jax.lax.erf(h * (1.0 / math.sqrt(2.0))))
        o = o + jnp.dot(h.astype(jnp.bfloat16),
                        w2_ref[0, ks, :].astype(jnp.bfloat16),
                        preferred_element_type=jnp.float32)
    contrib = w_e * o

    @pl.when(e == 0)
    def _():
        out_ref[sl, :] = contrib

    @pl.when(e != 0)
    def _():
        out_ref[sl, :] = out_ref[sl, :] + contrib


@jax.jit
def _soft_moe(x, Wr, br, W1, b1, W2, b2):
    B, S, D = x.shape
    E = Wr.shape[1]
    FF = W1.shape[2]
    TS = 512

    x2 = x.reshape(S, D).astype(jnp.bfloat16)
    br2 = br.reshape(1, E)
    b1r = b1.reshape(E, 1, FF)
    b2r = b2.reshape(E, 1, D)

    grid = (E, S // TS)
    out = pl.pallas_call(
        functools.partial(_moe_body, tile_s=TS, ff_chunk=FF // 4),
        grid=grid,
        in_specs=[
            pl.BlockSpec((S, D), lambda e, i: (0, 0)),
            pl.BlockSpec((D, E), lambda e, i: (0, 0)),
            pl.BlockSpec((1, E), lambda e, i: (0, 0)),
            pl.BlockSpec((1, D, FF), lambda e, i: (e, 0, 0)),
            pl.BlockSpec((1, 1, FF), lambda e, i: (e, 0, 0)),
            pl.BlockSpec((1, FF, D), lambda e, i: (e, 0, 0)),
            pl.BlockSpec((1, 1, D), lambda e, i: (e, 0, 0)),
        ],
        out_specs=pl.BlockSpec((S, D), lambda e, i: (0, 0)),
        out_shape=jax.ShapeDtypeStruct((S, D), jnp.float32),
    )(x2, Wr, br2, W1, b1r, W2, b2r)
    return out.reshape(B, S, D)


def kernel(x, Wr, br, W1, b1, W2, b2):
    return _soft_moe(x, Wr, br, W1, b1, W2, b2)


# speedup vs baseline: 4.5066x; 1.0614x over previous
import functools
import math

import jax
import jax.numpy as jnp
from jax.experimental import pallas as pl
from jax.experimental.pallas import tpu as pltpu


def _moe_body(x_ref, wr_ref, br_ref, w1_ref, b1_ref, w2_ref, b2_ref,
              out_ref, *, tile_s, ff_chunk):
    e = pl.program_id(0)
    i = pl.program_id(1)
    sl = pl.ds(i * tile_s, tile_s)

    xb = x_ref[sl, :]

    logits = jnp.dot(xb, wr_ref[...].astype(jnp.bfloat16),
                     preferred_element_type=jnp.float32)
    logits = logits + br_ref[...]
    logits = logits - jnp.max(logits, axis=-1, keepdims=True)
    p = jnp.exp(logits)
    weights = p / jnp.sum(p, axis=-1, keepdims=True)
    lane = jax.lax.broadcasted_iota(jnp.int32, weights.shape, 1)
    w_e = jnp.sum(jnp.where(lane == e, weights, 0.0), axis=1,
                  keepdims=True)

    ff = w1_ref.shape[2]
    n_chunks = ff // ff_chunk
    o = b2_ref[0].astype(jnp.float32)
    for k in range(n_chunks):
        ks = slice(k * ff_chunk, (k + 1) * ff_chunk)
        h = jnp.dot(xb, w1_ref[0, :, ks].astype(jnp.bfloat16),
                    preferred_element_type=jnp.float32)
        h = h + b1_ref[0, :, ks]
        h = 0.5 * h * (1.0 + jax.lax.erf(h * (1.0 / math.sqrt(2.0))))
        o = o + jnp.dot(h.astype(jnp.bfloat16),
                        w2_ref[0, ks, :].astype(jnp.bfloat16),
                        preferred_element_type=jnp.float32)
    contrib = w_e * o

    @pl.when(e == 0)
    def _():
        out_ref[sl, :] = contrib

    @pl.when(e != 0)
    def _():
        out_ref[sl, :] = out_ref[sl, :] + contrib


@jax.jit
def _soft_moe(x, Wr, br, W1, b1, W2, b2):
    B, S, D = x.shape
    E = Wr.shape[1]
    FF = W1.shape[2]
    TS = 1024

    x2 = x.reshape(S, D).astype(jnp.bfloat16)
    br2 = br.reshape(1, E)
    b1r = b1.reshape(E, 1, FF)
    b2r = b2.reshape(E, 1, D)

    grid = (E, S // TS)
    out = pl.pallas_call(
        functools.partial(_moe_body, tile_s=TS, ff_chunk=FF // 4),
        grid=grid,
        in_specs=[
            pl.BlockSpec((S, D), lambda e, i: (0, 0)),
            pl.BlockSpec((D, E), lambda e, i: (0, 0)),
            pl.BlockSpec((1, E), lambda e, i: (0, 0)),
            pl.BlockSpec((1, D, FF), lambda e, i: (e, 0, 0)),
            pl.BlockSpec((1, 1, FF), lambda e, i: (e, 0, 0)),
            pl.BlockSpec((1, FF, D), lambda e, i: (e, 0, 0)),
            pl.BlockSpec((1, 1, D), lambda e, i: (e, 0, 0)),
        ],
        out_specs=pl.BlockSpec((S, D), lambda e, i: (0, 0)),
        out_shape=jax.ShapeDtypeStruct((S, D), jnp.float32),
    )(x2, Wr, br2, W1, b1r, W2, b2r)
    return out.reshape(B, S, D)


def kernel(x, Wr, br, W1, b1, W2, b2):
    return _soft_moe(x, Wr, br, W1, b1, W2, b2)
